# baseline (speedup 1.0000x reference)
import jax
import jax.numpy as jnp
from jax.experimental import pallas as pl

_BATCH = 16
_PATCHES = 2048
_EMBED = 1024
_MASK_RATIO = 0.75
_NUM_MASKED = int(_MASK_RATIO * _PATCHES)

_ROWS_PER_BLK = 256


def _mask_kernel(x_ref, m_ref, o_ref):
    o_ref[0] = x_ref[0] * m_ref[0]


def kernel(x):
    pkey = jax.random.key(42)
    keys = jax.random.split(pkey, _BATCH)
    perms = jax.vmap(lambda k: jax.random.permutation(k, _PATCHES))(keys)
    mask_indices = perms[:, :_NUM_MASKED]
    keep = jnp.ones((_BATCH, _PATCHES), dtype=jnp.float32)
    keep = keep.at[jnp.arange(_BATCH)[:, None], mask_indices].set(0.0)
    keep3 = keep[:, :, None]
    n_blk = _PATCHES // _ROWS_PER_BLK
    masked_x = pl.pallas_call(
        _mask_kernel,
        grid=(_BATCH, n_blk),
        in_specs=[
            pl.BlockSpec((1, _ROWS_PER_BLK, _EMBED), lambda i, j: (i, j, 0)),
            pl.BlockSpec((1, _ROWS_PER_BLK, 1), lambda i, j: (i, j, 0)),
        ],
        out_specs=pl.BlockSpec((1, _ROWS_PER_BLK, _EMBED), lambda i, j: (i, j, 0)),
        out_shape=jax.ShapeDtypeStruct((_BATCH, _PATCHES, _EMBED), jnp.float32),
    )(x, keep3)
    return (masked_x, mask_indices, x)


# speedup vs baseline: 2.1440x; 2.1440x over previous
import functools

import jax
import jax.numpy as jnp
import numpy as np
from jax.experimental import pallas as pl

_BATCH = 16
_PATCHES = 2048
_EMBED = 1024
_MASK_RATIO = 0.75
_NUM_MASKED = int(_MASK_RATIO * _PATCHES)

_ROWS_PER_BLK = 256


@functools.lru_cache(maxsize=1)
def _static_mask():
    cpu = jax.local_devices(backend="cpu")[0]
    with jax.ensure_compile_time_eval(), jax.default_device(cpu):
        pkey = jax.random.key(42)
        keys = jax.random.split(pkey, _BATCH)
        perms = jax.vmap(lambda k: jax.random.permutation(k, _PATCHES))(keys)
        perms = np.asarray(perms)
    mask_indices = perms[:, :_NUM_MASKED].astype(np.int32)
    keep = np.ones((_BATCH, _PATCHES), dtype=np.float32)
    keep[np.arange(_BATCH)[:, None], mask_indices] = 0.0
    return mask_indices, keep


def _mask_kernel(x_ref, m_ref, o_ref, c_ref):
    xb = x_ref[0]
    o_ref[0] = xb * m_ref[0]
    c_ref[0] = xb


def kernel(x):
    mask_indices, keep = _static_mask()
    keep3 = jnp.asarray(keep.reshape(_BATCH, _PATCHES, 1))
    n_blk = _PATCHES // _ROWS_PER_BLK
    masked_x, x_original = pl.pallas_call(
        _mask_kernel,
        grid=(_BATCH, n_blk),
        in_specs=[
            pl.BlockSpec((1, _ROWS_PER_BLK, _EMBED), lambda i, j: (i, j, 0)),
            pl.BlockSpec((1, _ROWS_PER_BLK, 1), lambda i, j: (i, j, 0)),
        ],
        out_specs=[
            pl.BlockSpec((1, _ROWS_PER_BLK, _EMBED), lambda i, j: (i, j, 0)),
            pl.BlockSpec((1, _ROWS_PER_BLK, _EMBED), lambda i, j: (i, j, 0)),
        ],
        out_shape=[
            jax.ShapeDtypeStruct((_BATCH, _PATCHES, _EMBED), jnp.float32),
            jax.ShapeDtypeStruct((_BATCH, _PATCHES, _EMBED), jnp.float32),
        ],
    )(x, keep3)
    return (masked_x, jnp.asarray(mask_indices), x_original)


# speedup vs baseline: 2.5879x; 1.2071x over previous
import functools

import jax
import jax.numpy as jnp
import numpy as np
from jax.experimental import pallas as pl

_BATCH = 16
_PATCHES = 2048
_EMBED = 1024
_MASK_RATIO = 0.75
_NUM_MASKED = int(_MASK_RATIO * _PATCHES)

_ROWS_PER_BLK = 512


@functools.lru_cache(maxsize=1)
def _static_mask():
    cpu = jax.local_devices(backend="cpu")[0]
    with jax.ensure_compile_time_eval(), jax.default_device(cpu):
        pkey = jax.random.key(42)
        keys = jax.random.split(pkey, _BATCH)
        perms = jax.vmap(lambda k: jax.random.permutation(k, _PATCHES))(keys)
        perms = np.asarray(perms)
    mask_indices = perms[:, :_NUM_MASKED].astype(np.int32)
    keep = np.ones((_BATCH, _PATCHES), dtype=np.int8)
    keep[np.arange(_BATCH)[:, None], mask_indices] = 0
    return mask_indices, keep


def _mask_kernel(x_ref, m_ref, o_ref, c_ref):
    xb = x_ref[0]
    o_ref[0] = xb * m_ref[0].astype(jnp.float32)
    c_ref[0] = xb


def kernel(x):
    mask_indices, keep = _static_mask()
    keep3 = jnp.asarray(keep.reshape(_BATCH, _PATCHES, 1))
    n_blk = _PATCHES // _ROWS_PER_BLK
    masked_x, x_original = pl.pallas_call(
        _mask_kernel,
        grid=(_BATCH, n_blk),
        in_specs=[
            pl.BlockSpec((1, _ROWS_PER_BLK, _EMBED), lambda i, j: (i, j, 0)),
            pl.BlockSpec((1, _ROWS_PER_BLK, 1), lambda i, j: (i, j, 0)),
        ],
        out_specs=[
            pl.BlockSpec((1, _ROWS_PER_BLK, _EMBED), lambda i, j: (i, j, 0)),
            pl.BlockSpec((1, _ROWS_PER_BLK, _EMBED), lambda i, j: (i, j, 0)),
        ],
        out_shape=[
            jax.ShapeDtypeStruct((_BATCH, _PATCHES, _EMBED), jnp.float32),
            jax.ShapeDtypeStruct((_BATCH, _PATCHES, _EMBED), jnp.float32),
        ],
    )(x, keep3)
    return (masked_x, jnp.asarray(mask_indices), x_original)


# speedup vs baseline: 2.6985x; 1.0427x over previous
import functools

import jax
import jax.numpy as jnp
import numpy as np
from jax.experimental import pallas as pl

_BATCH = 16
_PATCHES = 2048
_EMBED = 1024
_MASK_RATIO = 0.75
_NUM_MASKED = int(_MASK_RATIO * _PATCHES)

_ROWS_PER_BLK = 1024


@functools.lru_cache(maxsize=1)
def _static_mask():
    cpu = jax.local_devices(backend="cpu")[0]
    with jax.ensure_compile_time_eval(), jax.default_device(cpu):
        pkey = jax.random.key(42)
        keys = jax.random.split(pkey, _BATCH)
        perms = jax.vmap(lambda k: jax.random.permutation(k, _PATCHES))(keys)
        perms = np.asarray(perms)
    mask_indices = perms[:, :_NUM_MASKED].astype(np.int32)
    keep = np.ones((_BATCH, _PATCHES), dtype=np.int8)
    keep[np.arange(_BATCH)[:, None], mask_indices] = 0
    return mask_indices, keep


def _mask_kernel(x_ref, m_ref, o_ref, c_ref):
    xb = x_ref[0]
    o_ref[0] = xb * m_ref[0].astype(jnp.float32)
    c_ref[0] = xb


def kernel(x):
    mask_indices, keep = _static_mask()
    keep3 = jnp.asarray(keep.reshape(_BATCH, _PATCHES, 1))
    n_blk = _PATCHES // _ROWS_PER_BLK
    masked_x, x_original = pl.pallas_call(
        _mask_kernel,
        grid=(_BATCH, n_blk),
        in_specs=[
            pl.BlockSpec((1, _ROWS_PER_BLK, _EMBED), lambda i, j: (i, j, 0)),
            pl.BlockSpec((1, _ROWS_PER_BLK, 1), lambda i, j: (i, j, 0)),
        ],
        out_specs=[
            pl.BlockSpec((1, _ROWS_PER_BLK, _EMBED), lambda i, j: (i, j, 0)),
            pl.BlockSpec((1, _ROWS_PER_BLK, _EMBED), lambda i, j: (i, j, 0)),
        ],
        out_shape=[
            jax.ShapeDtypeStruct((_BATCH, _PATCHES, _EMBED), jnp.float32),
            jax.ShapeDtypeStruct((_BATCH, _PATCHES, _EMBED), jnp.float32),
        ],
    )(x, keep3)
    return (masked_x, jnp.asarray(mask_indices), x_original)


# speedup vs baseline: 2.7764x; 1.0289x over previous
import functools

import jax
import jax.numpy as jnp
import numpy as np
from jax.experimental import pallas as pl

_BATCH = 16
_PATCHES = 2048
_EMBED = 1024
_MASK_RATIO = 0.75
_NUM_MASKED = int(_MASK_RATIO * _PATCHES)

_ROWS_PER_BLK = 2048


@functools.lru_cache(maxsize=1)
def _static_mask():
    cpu = jax.local_devices(backend="cpu")[0]
    with jax.ensure_compile_time_eval(), jax.default_device(cpu):
        pkey = jax.random.key(42)
        keys = jax.random.split(pkey, _BATCH)
        perms = jax.vmap(lambda k: jax.random.permutation(k, _PATCHES))(keys)
        perms = np.asarray(perms)
    mask_indices = perms[:, :_NUM_MASKED].astype(np.int32)
    keep = np.ones((_BATCH, _PATCHES), dtype=np.int8)
    keep[np.arange(_BATCH)[:, None], mask_indices] = 0
    return mask_indices, keep


def _mask_kernel(x_ref, m_ref, o_ref, c_ref):
    xb = x_ref[0]
    o_ref[0] = xb * m_ref[0].astype(jnp.float32)
    c_ref[0] = xb


def kernel(x):
    mask_indices, keep = _static_mask()
    keep3 = jnp.asarray(keep.reshape(_BATCH, _PATCHES, 1))
    n_blk = _PATCHES // _ROWS_PER_BLK
    masked_x, x_original = pl.pallas_call(
        _mask_kernel,
        grid=(_BATCH, n_blk),
        in_specs=[
            pl.BlockSpec((1, _ROWS_PER_BLK, _EMBED), lambda i, j: (i, j, 0)),
            pl.BlockSpec((1, _ROWS_PER_BLK, 1), lambda i, j: (i, j, 0)),
        ],
        out_specs=[
            pl.BlockSpec((1, _ROWS_PER_BLK, _EMBED), lambda i, j: (i, j, 0)),
            pl.BlockSpec((1, _ROWS_PER_BLK, _EMBED), lambda i, j: (i, j, 0)),
        ],
        out_shape=[
            jax.ShapeDtypeStruct((_BATCH, _PATCHES, _EMBED), jnp.float32),
            jax.ShapeDtypeStruct((_BATCH, _PATCHES, _EMBED), jnp.float32),
        ],
    )(x, keep3)
    return (masked_x, jnp.asarray(mask_indices), x_original)
